# baseline (speedup 1.0000x reference)
import functools

import jax
import jax.numpy as jnp
from jax import lax
from jax.experimental import pallas as pl
from jax.experimental.pallas import tpu as pltpu

AUG_P = 0.5
DROP_P = 0.15
MIN_FEATURES = 1
NRING = 5


def _fused(s_ref, m_ref, aug_ref, x_hbm, o_hbm, kv,
           i0, i1, i2, i3, i4, o0, o1, o2, o3, o4,
           si0, si1, si2, si3, si4, so0, so1, so2, so3, so4,
           *, B, F, C, FBLK):
    NBLK = F // FBLK
    NROUND = NBLK // NRING
    ibufs = (i0, i1, i2, i3, i4)
    obufs = (o0, o1, o2, o3, o4)
    sin = (si0, si1, si2, si3, si4)
    sout = (so0, so1, so2, so3, so4)

    def in_desc(p, j):
        return pltpu.make_async_copy(
            x_hbm.at[pl.ds(p * FBLK, FBLK)], ibufs[j], sin[j]
        )

    def out_desc(p, j):
        return pltpu.make_async_copy(
            obufs[j], o_hbm.at[pl.ds(p * FBLK, FBLK)], sout[j]
        )

    for q in range(NRING - 1):
        in_desc(q, q).start()

    m = m_ref[...].T > 0
    bits = lax.bitcast_convert_type(s_ref[...].T, jnp.int32)
    bits = jnp.where(m, bits, jnp.int32(0x7FFFFFFF))
    n_avail = jnp.sum(m.astype(jnp.int32), axis=0, keepdims=True)
    k = (n_avail.astype(jnp.float32) * DROP_P).astype(jnp.int32)
    k = jnp.minimum(k, n_avail - MIN_FEATURES)
    aug = aug_ref[...].T < AUG_P
    k = jnp.where((n_avail > MIN_FEATURES) & aug & (k > 0), k, 0)
    ans = jnp.zeros((1, B), jnp.int32)
    for bit in range(30, -1, -1):
        test = ans + jnp.int32(1 << bit)
        cnt = jnp.sum((bits < test).astype(jnp.int32), axis=0, keepdims=True)
        ans = jnp.where(cnt < k, test, ans)
    c_lt = jnp.sum((bits < ans).astype(jnp.int32), axis=0, keepdims=True)
    eq = bits == ans
    fi = lax.broadcasted_iota(jnp.int32, (F, F), 0)
    fj = lax.broadcasted_iota(jnp.int32, (F, F), 1)
    tril = (fj < fi).astype(jnp.float32)
    eq_before = jax.lax.dot(
        tril, eq.astype(jnp.float32), precision=jax.lax.Precision.HIGHEST
    ).astype(jnp.int32)
    drop = m & ((bits < ans) | (eq & ((c_lt + eq_before) < k)))
    kv[...] = 1.0 - drop.astype(jnp.float32)

    def round_body(t, carry):
        for j in range(NRING):
            p = t * NRING + j
            j2 = (j + NRING - 1) % NRING

            @pl.when(p + NRING - 1 < NBLK)
            def _():
                in_desc(p + NRING - 1, j2).start()

            in_desc(p, j).wait()

            @pl.when(t > 0)
            def _():
                out_desc(p - NRING, j).wait()

            sel = (
                lax.broadcasted_iota(jnp.int32, (FBLK, F), 1)
                == p * FBLK + lax.broadcasted_iota(jnp.int32, (FBLK, F), 0)
            ).astype(jnp.float32)
            kb = jax.lax.dot(
                sel, kv[...], precision=jax.lax.Precision.HIGHEST
            )
            obufs[j][...] = ibufs[j][...] * kb[:, :, None]
            out_desc(p, j).start()
        return carry

    lax.fori_loop(0, NROUND, round_body, 0)
    for j in range(NRING):
        out_desc(NBLK - NRING + j, j).wait()


def kernel(input_features, attention_mask):
    B, F, C = input_features.shape
    key = jax.random.key(42)
    k1, k2 = jax.random.split(key)
    aug_u = jax.random.uniform(k1, (B,)).reshape(B, 1)
    scores = jax.random.uniform(k2, (B, F))
    mask_i32 = attention_mask.astype(jnp.int32)

    xt = jnp.transpose(input_features, (1, 0, 2))

    FBLK = 10
    out_t = pl.pallas_call(
        functools.partial(_fused, B=B, F=F, C=C, FBLK=FBLK),
        compiler_params=pltpu.CompilerParams(
            vmem_limit_bytes=100 * 1024 * 1024,
        ),
        in_specs=[
            pl.BlockSpec((B, F), lambda: (0, 0)),
            pl.BlockSpec((B, F), lambda: (0, 0)),
            pl.BlockSpec((B, 1), lambda: (0, 0)),
            pl.BlockSpec(memory_space=pl.ANY),
        ],
        out_specs=pl.BlockSpec(memory_space=pl.ANY),
        out_shape=jax.ShapeDtypeStruct((F, B, C), input_features.dtype),
        scratch_shapes=(
            [pltpu.VMEM((F, B), jnp.float32)]
            + [pltpu.VMEM((FBLK, B, C), jnp.float32) for _ in range(10)]
            + [pltpu.SemaphoreType.DMA for _ in range(10)]
        ),
    )(scores, mask_i32, aug_u, xt)
    return jnp.transpose(out_t, (1, 0, 2))
